# Initial kernel scaffold; baseline (speedup 1.0000x reference)
import jax
import jax.numpy as jnp
from jax.experimental import pallas as pl


def kernel(hidden_states, positions, Wqkv, Wo, Wg, Wgate, Wup, Wdown, wpa, wpoa, wpm, wpom):
    raise NotImplementedError("write your pallas kernel here")


# speedup vs baseline: 1.0505x; 1.0505x over previous
import numpy as np
import jax
import jax.numpy as jnp
from jax.experimental import pallas as pl

B, S, D = 1, 2048, 2048
H, HKV, DH = 16, 8, 128
E, TOPK, DFF = 8, 2, 2048
SOFTCAP = 30.0
ATTN_MULT = 0.08838834764831845
HALF = DH // 2
NQKV = (H + 2 * HKV) * DH

_GELU_C = float(np.sqrt(2.0 / np.pi))
_BF = jnp.bfloat16


def _gelu(x):
    return 0.5 * x * (1.0 + jnp.tanh(_GELU_C * (x + 0.044715 * x * x * x)))


def _dot(a, b):
    return jnp.dot(a.astype(_BF), b.astype(_BF), preferred_element_type=jnp.float32)


_BS1 = 2048
_BC1 = 512


def _qkv_body(x_ref, r_ref, wpa_ref, wqkv_ref, o_ref):
    h = x_ref[...] * r_ref[...] * wpa_ref[...]
    o_ref[...] = _dot(h, wqkv_ref[...])


def _qkv_proj(x, r, wpa, Wqkv):
    return pl.pallas_call(
        _qkv_body,
        grid=(NQKV // _BC1,),
        in_specs=[
            pl.BlockSpec((S, D), lambda c: (0, 0)),
            pl.BlockSpec((S, 1), lambda c: (0, 0)),
            pl.BlockSpec((1, D), lambda c: (0, 0)),
            pl.BlockSpec((D, _BC1), lambda c: (0, c)),
        ],
        out_specs=pl.BlockSpec((S, _BC1), lambda c: (0, c)),
        out_shape=jax.ShapeDtypeStruct((S, NQKV), jnp.float32),
    )(x, r, wpa, Wqkv)


_BQ = 512


def _rope(seg, cos, sin):
    x1 = seg[:, :HALF]
    x2 = seg[:, HALF:]
    return jnp.concatenate([x1 * cos - x2 * sin, x2 * cos + x1 * sin], axis=-1)


def _scores_body(q_ref, k_ref, e_ref):
    qb = pl.program_id(1)
    row0 = qb * _BQ
    s = jax.lax.dot_general(q_ref[...].astype(_BF), k_ref[...].astype(_BF),
                            (((1,), (1,)), ((), ())),
                            preferred_element_type=jnp.float32)
    s = s * (DH ** -0.5)
    s = SOFTCAP * jnp.tanh(s / SOFTCAP)
    rows = row0 + jax.lax.broadcasted_iota(jnp.int32, (_BQ, S), 0)
    cols = jax.lax.broadcasted_iota(jnp.int32, (_BQ, S), 1)
    s = jnp.where(rows >= cols, s, -1e30)
    m = jnp.max(s, axis=-1, keepdims=True)
    e_ref[0] = jnp.exp(s - m)


def _scores(q_rot, k_rot):
    return pl.pallas_call(
        _scores_body,
        grid=(H, S // _BQ),
        in_specs=[
            pl.BlockSpec((_BQ, DH), lambda h, qb: (qb, h)),
            pl.BlockSpec((S, DH), lambda h, qb: (0, h // 2)),
        ],
        out_specs=pl.BlockSpec((1, _BQ, S), lambda h, qb: (h, qb, 0)),
        out_shape=jax.ShapeDtypeStruct((H, S, S), jnp.float32),
    )(q_rot, k_rot)


def _pv_body(e_ref, den_ref, v_ref, o_ref):
    p = e_ref[0] / den_ref[0]
    o_ref[...] = _dot(p, v_ref[...])


def _pv(Eexp, den, qkv):
    return pl.pallas_call(
        _pv_body,
        grid=(H,),
        in_specs=[
            pl.BlockSpec((1, S, S), lambda h: (h, 0, 0)),
            pl.BlockSpec((1, S, 1), lambda h: (h, 0, 0)),
            pl.BlockSpec((S, DH), lambda h: (0, H + HKV + h // 2)),
        ],
        out_specs=pl.BlockSpec((S, DH), lambda h: (0, h)),
        out_shape=jax.ShapeDtypeStruct((S, H * DH), jnp.float32),
    )(Eexp, den, qkv)


_BS3 = 256


def _oproj_body(a_ref, wo_ref, o_ref):
    o_ref[...] = _dot(a_ref[...], wo_ref[...]) * ATTN_MULT


_BCO = 512


def _oproj(attn, Wo):
    return pl.pallas_call(
        _oproj_body,
        grid=(D // _BCO,),
        in_specs=[
            pl.BlockSpec((S, H * DH), lambda c: (0, 0)),
            pl.BlockSpec((H * DH, _BCO), lambda c: (0, c)),
        ],
        out_specs=pl.BlockSpec((S, _BCO), lambda c: (0, c)),
        out_shape=jax.ShapeDtypeStruct((S, D), jnp.float32),
    )(attn, Wo)


_EPAD = 128


def _router_body(xt_ref, wg_ref, tw_ref):
    l = _dot(xt_ref[...], wg_ref[...])
    l = SOFTCAP * jnp.tanh(l / SOFTCAP)
    lane = jax.lax.broadcasted_iota(jnp.int32, l.shape, 1)
    valid = lane < E
    lm = jnp.where(valid, l, -jnp.inf)
    m = jnp.max(lm, axis=-1, keepdims=True)
    ex = jnp.exp(lm - m)
    p = ex / jnp.sum(ex, axis=-1, keepdims=True)
    m1 = jnp.max(p, axis=-1, keepdims=True)
    a1 = jnp.min(jnp.where(p == m1, lane, _EPAD), axis=-1, keepdims=True)
    sel1 = lane == a1
    p2 = jnp.where(sel1, -1.0, p)
    m2 = jnp.max(p2, axis=-1, keepdims=True)
    a2 = jnp.min(jnp.where(p2 == m2, lane, _EPAD), axis=-1, keepdims=True)
    sel2 = lane == a2
    tw_ref[...] = jnp.where(sel1 | sel2, p / (m1 + m2), 0.0)


def _router(xt, Wg):
    wg_pad = jnp.pad(Wg, ((0, 0), (0, _EPAD - E)))
    return pl.pallas_call(
        _router_body,
        grid=(1,),
        in_specs=[
            pl.BlockSpec((S, D), lambda s: (0, 0)),
            pl.BlockSpec((D, _EPAD), lambda s: (0, 0)),
        ],
        out_specs=pl.BlockSpec((S, _EPAD), lambda s: (0, 0)),
        out_shape=jax.ShapeDtypeStruct((S, _EPAD), jnp.float32),
    )(xt, wg_pad)


_BS5 = 256
_BF5 = 512


def _moe_body(xt_ref, tw_ref, wg_ref, wu_ref, wd_ref, o_ref):
    e = pl.program_id(0)
    f = pl.program_id(1)
    sb = pl.program_id(2)
    xt = xt_ref[...]
    g = _gelu(_dot(xt, wg_ref[0]))
    u = _dot(xt, wu_ref[0])
    contrib = _dot(g * u, wd_ref[0])
    lane = jax.lax.broadcasted_iota(jnp.int32, (_BS5, _EPAD), 1)
    te = jnp.sum(jnp.where(lane == e, tw_ref[...], 0.0), axis=-1, keepdims=True)
    contrib = contrib * te
    row0 = pl.multiple_of(sb * _BS5, _BS5)
    @pl.when(jnp.logical_and(e == 0, f == 0))
    def _init():
        o_ref[pl.ds(row0, _BS5), :] = contrib
    @pl.when(jnp.logical_not(jnp.logical_and(e == 0, f == 0)))
    def _acc():
        o_ref[pl.ds(row0, _BS5), :] += contrib


def _moe(xt, tw, Wgate, Wup, Wdown):
    return pl.pallas_call(
        _moe_body,
        grid=(E, DFF // _BF5, S // _BS5),
        in_specs=[
            pl.BlockSpec((_BS5, D), lambda e, f, sb: (sb, 0)),
            pl.BlockSpec((_BS5, _EPAD), lambda e, f, sb: (sb, 0)),
            pl.BlockSpec((1, D, _BF5), lambda e, f, sb: (e, 0, f)),
            pl.BlockSpec((1, D, _BF5), lambda e, f, sb: (e, 0, f)),
            pl.BlockSpec((1, _BF5, D), lambda e, f, sb: (e, f, 0)),
        ],
        out_specs=pl.BlockSpec((S, D), lambda e, f, sb: (0, 0)),
        out_shape=jax.ShapeDtypeStruct((S, D), jnp.float32),
    )(xt, tw, Wgate, Wup, Wdown)


def _final_body(h1_ref, moe_ref, rf_ref, wpom_ref, o_ref):
    o_ref[...] = h1_ref[...] + moe_ref[...] * rf_ref[...] * wpom_ref[...]


def _final(h1, moe, rf, wpom):
    return pl.pallas_call(
        _final_body,
        grid=(S // _BS3,),
        in_specs=[
            pl.BlockSpec((_BS3, D), lambda s: (s, 0)),
            pl.BlockSpec((_BS3, D), lambda s: (s, 0)),
            pl.BlockSpec((_BS3, 1), lambda s: (s, 0)),
            pl.BlockSpec((1, D), lambda s: (0, 0)),
        ],
        out_specs=pl.BlockSpec((_BS3, D), lambda s: (s, 0)),
        out_shape=jax.ShapeDtypeStruct((S, D), jnp.float32),
    )(h1, moe, rf, wpom)


def _rms_scale(x2d):
    x3 = x2d.reshape(1, S, D)
    r = jax.lax.rsqrt(jnp.mean(x3 * x3, axis=-1, keepdims=True) + 1e-5)
    return r.reshape(S, 1)


def kernel(hidden_states, positions, Wqkv, Wo, Wg, Wgate, Wup, Wdown,
           wpa, wpoa, wpm, wpom):
    x = hidden_states.reshape(S, D)
    pos = positions.reshape(S).astype(jnp.float32)
    inv_freq = 1.0 / (10000.0 ** (jnp.arange(HALF, dtype=jnp.float32) / HALF))
    ang = pos[:, None] * inv_freq
    cos = jnp.cos(ang)
    sin = jnp.sin(ang)

    r1 = _rms_scale(x)
    qkv = _qkv_proj(x, r1, wpa.reshape(1, D), Wqkv)
    cos4 = cos[None, :, None, :]
    sin4 = sin[None, :, None, :]
    q4 = qkv[:, :H * DH].reshape(1, S, H, DH)
    k4 = qkv[:, H * DH:H * DH + HKV * DH].reshape(1, S, HKV, DH)
    def rot(t):
        t1, t2 = t[..., :HALF], t[..., HALF:]
        return jnp.concatenate([t1 * cos4 - t2 * sin4, t2 * cos4 + t1 * sin4], axis=-1)
    q_rot = rot(q4).reshape(S, H * DH)
    k_rot = rot(k4).reshape(S, HKV * DH)
    Eexp = _scores(q_rot, k_rot)
    den = jnp.sum(Eexp.reshape(1, H, S, S), axis=-1).reshape(H, S, 1)
    attn = _pv(Eexp, den, qkv)
    ao = _oproj(attn, Wo)
    ra = _rms_scale(ao)
    h1 = x + ao * ra * wpoa.reshape(1, D)
    rm = _rms_scale(h1)
    xt = h1 * rm * wpm.reshape(1, D)
    tw = _router(xt, Wg)
    moe = _moe(xt, tw, Wgate, Wup, Wdown)
    rf = _rms_scale(moe)
    out = _final(h1, moe, rf, wpom.reshape(1, D))
    return out.reshape(B, S, D)
